# baseline (speedup 1.0000x reference)
import jax
import jax.numpy as jnp
from jax.experimental import pallas as pl
from jax.experimental.pallas import tpu as pltpu

E = 8
TOPK = 2
D = 1024
DFF = 512
T = 2048
BT = 256
NTB = T // BT


def _router_body(x_ref, rw_ref, cb_ref, comb_ref):
    x = x_ref[...]
    logits = jnp.dot(x, rw_ref[...], preferred_element_type=jnp.float32)
    m = jnp.max(logits, axis=-1, keepdims=True)
    ex = jnp.exp(logits - m)
    scores = ex / jnp.sum(ex, axis=-1, keepdims=True)
    b = scores + cb_ref[...]
    ids = jax.lax.broadcasted_iota(jnp.int32, (T, E), 1)
    m1 = jnp.max(b, axis=-1, keepdims=True)
    i1 = jnp.min(jnp.where(b == m1, ids, E), axis=-1, keepdims=True)
    b2 = jnp.where(ids == i1, -1e30, b)
    m2 = jnp.max(b2, axis=-1, keepdims=True)
    i2 = jnp.min(jnp.where(b2 == m2, ids, E), axis=-1, keepdims=True)
    w1 = jnp.sum(jnp.where(ids == i1, scores, 0.0), axis=-1, keepdims=True)
    w2 = jnp.sum(jnp.where(ids == i2, scores, 0.0), axis=-1, keepdims=True)
    comb_ref[...] = jnp.where(ids == i1, w1, 0.0) + jnp.where(ids == i2, w2, 0.0)


def _moe_body(comb_ref, x_ref, wgu_ref, wd_ref, out_ref, xb_ref, wgub_ref, wdb_ref):
    e = pl.program_id(0)
    tb = pl.program_id(1)
    row0 = pl.multiple_of(tb * BT, BT)

    @pl.when(jnp.logical_and(e == 0, tb == 0))
    def _cast_x():
        xb_ref[...] = x_ref[...].astype(jnp.bfloat16)

    @pl.when(tb == 0)
    def _cast_w():
        wgub_ref[...] = wgu_ref[0].astype(jnp.bfloat16)
        wdb_ref[...] = wd_ref[0].astype(jnp.bfloat16)

    x = xb_ref[pl.ds(row0, BT), :]
    gu = jnp.dot(x, wgub_ref[...], preferred_element_type=jnp.float32)
    gate = gu[:, :DFF]
    up = gu[:, DFF:]
    h = (gate * jax.lax.logistic(gate) * up).astype(jnp.bfloat16)
    y = jnp.dot(h, wdb_ref[...], preferred_element_type=jnp.float32)
    cslice = comb_ref[pl.ds(row0, BT), :]
    c = jnp.zeros((BT, 1), jnp.float32)
    for j in range(E):
        c = c + jnp.where(e == j, cslice[:, j:j + 1], 0.0)
    contrib = y * c

    @pl.when(e == 0)
    def _init():
        out_ref[pl.ds(row0, BT), :] = contrib

    @pl.when(e != 0)
    def _acc():
        out_ref[pl.ds(row0, BT), :] += contrib


def kernel(hidden_states, router_w, correction_bias, w_gate_up, w_down):
    cb2 = correction_bias.reshape(1, E)
    comb = pl.pallas_call(
        _router_body,
        out_shape=jax.ShapeDtypeStruct((T, E), jnp.float32),
    )(hidden_states, router_w, cb2)

    out = pl.pallas_call(
        _moe_body,
        grid=(E, NTB),
        in_specs=[
            pl.BlockSpec((T, E), lambda e, tb: (0, 0)),
            pl.BlockSpec((T, D), lambda e, tb: (0, 0)),
            pl.BlockSpec((1, D, 2 * DFF), lambda e, tb: (e, 0, 0)),
            pl.BlockSpec((1, DFF, D), lambda e, tb: (e, 0, 0)),
        ],
        out_specs=pl.BlockSpec((T, D), lambda e, tb: (0, 0)),
        out_shape=jax.ShapeDtypeStruct((T, D), jnp.float32),
        scratch_shapes=[
            pltpu.VMEM((T, D), jnp.bfloat16),
            pltpu.VMEM((D, 2 * DFF), jnp.bfloat16),
            pltpu.VMEM((DFF, D), jnp.bfloat16),
        ],
    )(comb, hidden_states, w_gate_up, w_down)
    return out


# speedup vs baseline: 1.2151x; 1.2151x over previous
import jax
import jax.numpy as jnp
from jax.experimental import pallas as pl
from jax.experimental.pallas import tpu as pltpu

E = 8
TOPK = 2
D = 1024
DFF = 512
T = 2048
BT = 512
NTB = T // BT


def _router_body(x_ref, rw_ref, cb_ref, comb_ref):
    x = x_ref[...]
    logits = jnp.dot(x, rw_ref[...], preferred_element_type=jnp.float32)
    m = jnp.max(logits, axis=-1, keepdims=True)
    ex = jnp.exp(logits - m)
    scores = ex / jnp.sum(ex, axis=-1, keepdims=True)
    b = scores + cb_ref[...]
    ids = jax.lax.broadcasted_iota(jnp.int32, (T, E), 1)
    m1 = jnp.max(b, axis=-1, keepdims=True)
    i1 = jnp.min(jnp.where(b == m1, ids, E), axis=-1, keepdims=True)
    b2 = jnp.where(ids == i1, -1e30, b)
    m2 = jnp.max(b2, axis=-1, keepdims=True)
    i2 = jnp.min(jnp.where(b2 == m2, ids, E), axis=-1, keepdims=True)
    w1 = jnp.sum(jnp.where(ids == i1, scores, 0.0), axis=-1, keepdims=True)
    w2 = jnp.sum(jnp.where(ids == i2, scores, 0.0), axis=-1, keepdims=True)
    comb_ref[...] = jnp.where(ids == i1, w1, 0.0) + jnp.where(ids == i2, w2, 0.0)


def _moe_body(comb_ref, x_ref, wgu_ref, wd_ref, out_ref, xb_ref, wgub_ref, wdb_ref):
    e = pl.program_id(0)
    tb = pl.program_id(1)
    row0 = pl.multiple_of(tb * BT, BT)

    @pl.when(jnp.logical_and(e == 0, tb == 0))
    def _cast_x():
        xb_ref[...] = x_ref[...].astype(jnp.bfloat16)

    @pl.when(tb == 0)
    def _cast_w():
        wgub_ref[...] = wgu_ref[0].astype(jnp.bfloat16)
        wdb_ref[...] = wd_ref[0].astype(jnp.bfloat16)

    x = xb_ref[pl.ds(row0, BT), :]
    gu = jnp.dot(x, wgub_ref[...], preferred_element_type=jnp.float32)
    gate = gu[:, :DFF]
    up = gu[:, DFF:]
    h = (gate * jax.lax.logistic(gate) * up).astype(jnp.bfloat16)
    y = jnp.dot(h, wdb_ref[...], preferred_element_type=jnp.float32)
    cslice = comb_ref[pl.ds(row0, BT), :]
    c = jnp.zeros((BT, 1), jnp.float32)
    for j in range(E):
        c = c + jnp.where(e == j, cslice[:, j:j + 1], 0.0)
    contrib = y * c

    @pl.when(e == 0)
    def _init():
        out_ref[pl.ds(row0, BT), :] = contrib

    @pl.when(e != 0)
    def _acc():
        out_ref[pl.ds(row0, BT), :] += contrib


def kernel(hidden_states, router_w, correction_bias, w_gate_up, w_down):
    cb2 = correction_bias.reshape(1, E)
    comb = pl.pallas_call(
        _router_body,
        out_shape=jax.ShapeDtypeStruct((T, E), jnp.float32),
    )(hidden_states, router_w, cb2)

    out = pl.pallas_call(
        _moe_body,
        grid=(E, NTB),
        in_specs=[
            pl.BlockSpec((T, E), lambda e, tb: (0, 0)),
            pl.BlockSpec((T, D), lambda e, tb: (0, 0)),
            pl.BlockSpec((1, D, 2 * DFF), lambda e, tb: (e, 0, 0)),
            pl.BlockSpec((1, DFF, D), lambda e, tb: (e, 0, 0)),
        ],
        out_specs=pl.BlockSpec((T, D), lambda e, tb: (0, 0)),
        out_shape=jax.ShapeDtypeStruct((T, D), jnp.float32),
        scratch_shapes=[
            pltpu.VMEM((T, D), jnp.bfloat16),
            pltpu.VMEM((D, 2 * DFF), jnp.bfloat16),
            pltpu.VMEM((DFF, D), jnp.bfloat16),
        ],
    )(comb, hidden_states, w_gate_up, w_down)
    return out


# speedup vs baseline: 1.3190x; 1.0855x over previous
import jax
import jax.numpy as jnp
from jax.experimental import pallas as pl
from jax.experimental.pallas import tpu as pltpu

E = 8
TOPK = 2
D = 1024
DFF = 512
T = 2048
BT = 1024
NTB = T // BT


def _router_body(x_ref, rw_ref, cb_ref, comb_ref):
    x = x_ref[...]
    logits = jnp.dot(x, rw_ref[...], preferred_element_type=jnp.float32)
    m = jnp.max(logits, axis=-1, keepdims=True)
    ex = jnp.exp(logits - m)
    scores = ex / jnp.sum(ex, axis=-1, keepdims=True)
    b = scores + cb_ref[...]
    ids = jax.lax.broadcasted_iota(jnp.int32, (T, E), 1)
    m1 = jnp.max(b, axis=-1, keepdims=True)
    i1 = jnp.min(jnp.where(b == m1, ids, E), axis=-1, keepdims=True)
    b2 = jnp.where(ids == i1, -1e30, b)
    m2 = jnp.max(b2, axis=-1, keepdims=True)
    i2 = jnp.min(jnp.where(b2 == m2, ids, E), axis=-1, keepdims=True)
    w1 = jnp.sum(jnp.where(ids == i1, scores, 0.0), axis=-1, keepdims=True)
    w2 = jnp.sum(jnp.where(ids == i2, scores, 0.0), axis=-1, keepdims=True)
    comb_ref[...] = jnp.where(ids == i1, w1, 0.0) + jnp.where(ids == i2, w2, 0.0)


def _moe_body(comb_ref, x_ref, wgu_ref, wd_ref, out_ref, xb_ref, wgub_ref, wdb_ref):
    e = pl.program_id(0)
    tb = pl.program_id(1)
    row0 = pl.multiple_of(tb * BT, BT)

    @pl.when(jnp.logical_and(e == 0, tb == 0))
    def _cast_x():
        xb_ref[...] = x_ref[...].astype(jnp.bfloat16)

    @pl.when(tb == 0)
    def _cast_w():
        wgub_ref[...] = wgu_ref[0].astype(jnp.bfloat16)
        wdb_ref[...] = wd_ref[0].astype(jnp.bfloat16)

    x = xb_ref[pl.ds(row0, BT), :]
    gu = jnp.dot(x, wgub_ref[...], preferred_element_type=jnp.float32)
    gate = gu[:, :DFF]
    up = gu[:, DFF:]
    h = (gate * jax.lax.logistic(gate) * up).astype(jnp.bfloat16)
    y = jnp.dot(h, wdb_ref[...], preferred_element_type=jnp.float32)
    cslice = comb_ref[pl.ds(row0, BT), :]
    c = jnp.zeros((BT, 1), jnp.float32)
    for j in range(E):
        c = c + jnp.where(e == j, cslice[:, j:j + 1], 0.0)
    contrib = y * c

    @pl.when(e == 0)
    def _init():
        out_ref[pl.ds(row0, BT), :] = contrib

    @pl.when(e != 0)
    def _acc():
        out_ref[pl.ds(row0, BT), :] += contrib


def kernel(hidden_states, router_w, correction_bias, w_gate_up, w_down):
    cb2 = correction_bias.reshape(1, E)
    comb = pl.pallas_call(
        _router_body,
        out_shape=jax.ShapeDtypeStruct((T, E), jnp.float32),
    )(hidden_states, router_w, cb2)

    out = pl.pallas_call(
        _moe_body,
        grid=(E, NTB),
        in_specs=[
            pl.BlockSpec((T, E), lambda e, tb: (0, 0)),
            pl.BlockSpec((T, D), lambda e, tb: (0, 0)),
            pl.BlockSpec((1, D, 2 * DFF), lambda e, tb: (e, 0, 0)),
            pl.BlockSpec((1, DFF, D), lambda e, tb: (e, 0, 0)),
        ],
        out_specs=pl.BlockSpec((T, D), lambda e, tb: (0, 0)),
        out_shape=jax.ShapeDtypeStruct((T, D), jnp.float32),
        scratch_shapes=[
            pltpu.VMEM((T, D), jnp.bfloat16),
            pltpu.VMEM((D, 2 * DFF), jnp.bfloat16),
            pltpu.VMEM((DFF, D), jnp.bfloat16),
        ],
    )(comb, hidden_states, w_gate_up, w_down)
    return out


# speedup vs baseline: 1.3467x; 1.0209x over previous
import jax
import jax.numpy as jnp
from jax.experimental import pallas as pl
from jax.experimental.pallas import tpu as pltpu

E = 8
TOPK = 2
D = 1024
DFF = 512
T = 2048
BT = 2048
NTB = T // BT


def _router_body(x_ref, rw_ref, cb_ref, comb_ref):
    x = x_ref[...]
    logits = jnp.dot(x, rw_ref[...], preferred_element_type=jnp.float32)
    m = jnp.max(logits, axis=-1, keepdims=True)
    ex = jnp.exp(logits - m)
    scores = ex / jnp.sum(ex, axis=-1, keepdims=True)
    b = scores + cb_ref[...]
    ids = jax.lax.broadcasted_iota(jnp.int32, (T, E), 1)
    m1 = jnp.max(b, axis=-1, keepdims=True)
    i1 = jnp.min(jnp.where(b == m1, ids, E), axis=-1, keepdims=True)
    b2 = jnp.where(ids == i1, -1e30, b)
    m2 = jnp.max(b2, axis=-1, keepdims=True)
    i2 = jnp.min(jnp.where(b2 == m2, ids, E), axis=-1, keepdims=True)
    w1 = jnp.sum(jnp.where(ids == i1, scores, 0.0), axis=-1, keepdims=True)
    w2 = jnp.sum(jnp.where(ids == i2, scores, 0.0), axis=-1, keepdims=True)
    comb_ref[...] = jnp.where(ids == i1, w1, 0.0) + jnp.where(ids == i2, w2, 0.0)


def _moe_body(comb_ref, x_ref, wgu_ref, wd_ref, out_ref, xb_ref, wgub_ref, wdb_ref):
    e = pl.program_id(0)
    tb = pl.program_id(1)
    row0 = pl.multiple_of(tb * BT, BT)

    @pl.when(jnp.logical_and(e == 0, tb == 0))
    def _cast_x():
        xb_ref[...] = x_ref[...].astype(jnp.bfloat16)

    @pl.when(tb == 0)
    def _cast_w():
        wgub_ref[...] = wgu_ref[0].astype(jnp.bfloat16)
        wdb_ref[...] = wd_ref[0].astype(jnp.bfloat16)

    x = xb_ref[pl.ds(row0, BT), :]
    gu = jnp.dot(x, wgub_ref[...], preferred_element_type=jnp.float32)
    gate = gu[:, :DFF]
    up = gu[:, DFF:]
    h = (gate * jax.lax.logistic(gate) * up).astype(jnp.bfloat16)
    y = jnp.dot(h, wdb_ref[...], preferred_element_type=jnp.float32)
    cslice = comb_ref[pl.ds(row0, BT), :]
    c = jnp.zeros((BT, 1), jnp.float32)
    for j in range(E):
        c = c + jnp.where(e == j, cslice[:, j:j + 1], 0.0)
    contrib = y * c

    @pl.when(e == 0)
    def _init():
        out_ref[pl.ds(row0, BT), :] = contrib

    @pl.when(e != 0)
    def _acc():
        out_ref[pl.ds(row0, BT), :] += contrib


def kernel(hidden_states, router_w, correction_bias, w_gate_up, w_down):
    cb2 = correction_bias.reshape(1, E)
    comb = pl.pallas_call(
        _router_body,
        out_shape=jax.ShapeDtypeStruct((T, E), jnp.float32),
    )(hidden_states, router_w, cb2)

    out = pl.pallas_call(
        _moe_body,
        grid=(E, NTB),
        in_specs=[
            pl.BlockSpec((T, E), lambda e, tb: (0, 0)),
            pl.BlockSpec((T, D), lambda e, tb: (0, 0)),
            pl.BlockSpec((1, D, 2 * DFF), lambda e, tb: (e, 0, 0)),
            pl.BlockSpec((1, DFF, D), lambda e, tb: (e, 0, 0)),
        ],
        out_specs=pl.BlockSpec((T, D), lambda e, tb: (0, 0)),
        out_shape=jax.ShapeDtypeStruct((T, D), jnp.float32),
        scratch_shapes=[
            pltpu.VMEM((T, D), jnp.bfloat16),
            pltpu.VMEM((D, 2 * DFF), jnp.bfloat16),
            pltpu.VMEM((DFF, D), jnp.bfloat16),
        ],
    )(comb, hidden_states, w_gate_up, w_down)
    return out


# speedup vs baseline: 1.3975x; 1.0378x over previous
import jax
import jax.numpy as jnp
from jax.experimental import pallas as pl
from jax.experimental.pallas import tpu as pltpu

E = 8
TOPK = 2
D = 1024
DFF = 512
T = 2048


def _router_body(x_ref, rw_ref, cb_ref, comb_ref, xb_ref):
    x = x_ref[...]
    xb_ref[...] = x.astype(jnp.bfloat16)
    logits = jnp.dot(x, rw_ref[...], preferred_element_type=jnp.float32)
    m = jnp.max(logits, axis=-1, keepdims=True)
    ex = jnp.exp(logits - m)
    scores = ex / jnp.sum(ex, axis=-1, keepdims=True)
    b = scores + cb_ref[...]
    ids = jax.lax.broadcasted_iota(jnp.int32, (T, E), 1)
    m1 = jnp.max(b, axis=-1, keepdims=True)
    i1 = jnp.min(jnp.where(b == m1, ids, E), axis=-1, keepdims=True)
    b2 = jnp.where(ids == i1, -1e30, b)
    m2 = jnp.max(b2, axis=-1, keepdims=True)
    i2 = jnp.min(jnp.where(b2 == m2, ids, E), axis=-1, keepdims=True)
    w1 = jnp.sum(jnp.where(ids == i1, scores, 0.0), axis=-1, keepdims=True)
    w2 = jnp.sum(jnp.where(ids == i2, scores, 0.0), axis=-1, keepdims=True)
    comb_ref[...] = jnp.where(ids == i1, w1, 0.0) + jnp.where(ids == i2, w2, 0.0)


def _moe_body(comb_ref, xb_ref, wgu_ref, wd_ref, out_ref):
    e = pl.program_id(0)
    x = xb_ref[...]
    wgu = wgu_ref[0].astype(jnp.bfloat16)
    gu = jnp.dot(x, wgu, preferred_element_type=jnp.float32)
    gate = gu[:, :DFF]
    up = gu[:, DFF:]
    cslice = comb_ref[...]
    c = jnp.zeros((T, 1), jnp.float32)
    for j in range(E):
        c = c + jnp.where(e == j, cslice[:, j:j + 1], 0.0)
    hw = (gate * jax.lax.logistic(gate) * up * c).astype(jnp.bfloat16)
    wd = wd_ref[0].astype(jnp.bfloat16)
    y = jnp.dot(hw, wd, preferred_element_type=jnp.float32)

    @pl.when(e == 0)
    def _init():
        out_ref[...] = y

    @pl.when(e != 0)
    def _acc():
        out_ref[...] += y


def kernel(hidden_states, router_w, correction_bias, w_gate_up, w_down):
    cb2 = correction_bias.reshape(1, E)
    comb, xb = pl.pallas_call(
        _router_body,
        out_shape=(
            jax.ShapeDtypeStruct((T, E), jnp.float32),
            jax.ShapeDtypeStruct((T, D), jnp.bfloat16),
        ),
    )(hidden_states, router_w, cb2)

    out = pl.pallas_call(
        _moe_body,
        grid=(E,),
        in_specs=[
            pl.BlockSpec((T, E), lambda e: (0, 0)),
            pl.BlockSpec((T, D), lambda e: (0, 0)),
            pl.BlockSpec((1, D, 2 * DFF), lambda e: (e, 0, 0)),
            pl.BlockSpec((1, DFF, D), lambda e: (e, 0, 0)),
        ],
        out_specs=pl.BlockSpec((T, D), lambda e: (0, 0)),
        out_shape=jax.ShapeDtypeStruct((T, D), jnp.float32),
    )(comb, xb, w_gate_up, w_down)
    return out


# speedup vs baseline: 1.4576x; 1.0430x over previous
import jax
import jax.numpy as jnp
from jax.experimental import pallas as pl
from jax.experimental.pallas import tpu as pltpu

E = 8
TOPK = 2
D = 1024
DFF = 512
T = 2048


def _body(x_ref, rw_ref, cb_ref, wgu_ref, wd_ref, out_ref, xb_ref, comb_ref):
    e = pl.program_id(0)

    @pl.when(e == 0)
    def _router():
        x = x_ref[...]
        xb_ref[...] = x.astype(jnp.bfloat16)
        logits = jnp.dot(x, rw_ref[...], preferred_element_type=jnp.float32)
        m = jnp.max(logits, axis=-1, keepdims=True)
        ex = jnp.exp(logits - m)
        scores = ex / jnp.sum(ex, axis=-1, keepdims=True)
        b = scores + cb_ref[...]
        ids = jax.lax.broadcasted_iota(jnp.int32, (T, E), 1)
        m1 = jnp.max(b, axis=-1, keepdims=True)
        i1 = jnp.min(jnp.where(b == m1, ids, E), axis=-1, keepdims=True)
        b2 = jnp.where(ids == i1, -1e30, b)
        m2 = jnp.max(b2, axis=-1, keepdims=True)
        i2 = jnp.min(jnp.where(b2 == m2, ids, E), axis=-1, keepdims=True)
        w1 = jnp.sum(jnp.where(ids == i1, scores, 0.0), axis=-1, keepdims=True)
        w2 = jnp.sum(jnp.where(ids == i2, scores, 0.0), axis=-1, keepdims=True)
        comb_ref[...] = jnp.where(ids == i1, w1, 0.0) + jnp.where(ids == i2, w2, 0.0)

    x = xb_ref[...]
    wgu = wgu_ref[0].astype(jnp.bfloat16)
    gu = jnp.dot(x, wgu, preferred_element_type=jnp.float32)
    gate = gu[:, :DFF]
    up = gu[:, DFF:]
    cslice = comb_ref[...]
    c = jnp.zeros((T, 1), jnp.float32)
    for j in range(E):
        c = c + jnp.where(e == j, cslice[:, j:j + 1], 0.0)
    hw = (gate * jax.lax.logistic(gate) * up * c).astype(jnp.bfloat16)
    wd = wd_ref[0].astype(jnp.bfloat16)
    y = jnp.dot(hw, wd, preferred_element_type=jnp.float32)

    @pl.when(e == 0)
    def _init():
        out_ref[...] = y

    @pl.when(e != 0)
    def _acc():
        out_ref[...] += y


def kernel(hidden_states, router_w, correction_bias, w_gate_up, w_down):
    cb2 = correction_bias.reshape(1, E)
    out = pl.pallas_call(
        _body,
        grid=(E,),
        in_specs=[
            pl.BlockSpec((T, D), lambda e: (0, 0)),
            pl.BlockSpec((D, E), lambda e: (0, 0)),
            pl.BlockSpec((1, E), lambda e: (0, 0)),
            pl.BlockSpec((1, D, 2 * DFF), lambda e: (e, 0, 0)),
            pl.BlockSpec((1, DFF, D), lambda e: (e, 0, 0)),
        ],
        out_specs=pl.BlockSpec((T, D), lambda e: (0, 0)),
        out_shape=jax.ShapeDtypeStruct((T, D), jnp.float32),
        scratch_shapes=[
            pltpu.VMEM((T, D), jnp.bfloat16),
            pltpu.VMEM((T, E), jnp.float32),
        ],
    )(hidden_states, router_w, cb2, w_gate_up, w_down)
    return out
